# Initial kernel scaffold; baseline (speedup 1.0000x reference)
import jax
import jax.numpy as jnp
from jax import lax
from jax.experimental import pallas as pl
from jax.experimental.pallas import tpu as pltpu
from jax.experimental.pallas import tpu_sc as plsc

N = 100000
D = 128
H = 128
K = 16
ALPHA = 0.2

_NC, _NS = 2, 16
_NW = _NC * _NS
_RPW = 3128
_NPAD = _NW * _RPW

_ROWS = 5000


def _dense_body(f_ref, c_ref, w_ref, v_ref, a1_ref, a2_ref):
    wv = jnp.dot(w_ref[...], v_ref[...], preferred_element_type=jnp.float32)
    m1 = jnp.dot(c_ref[...], wv[:H], preferred_element_type=jnp.float32)
    m2 = jnp.dot(c_ref[...], wv[H:], preferred_element_type=jnp.float32)
    x = f_ref[...]
    a1_ref[...] = jnp.dot(x, m1, preferred_element_type=jnp.float32)
    a2_ref[...] = jnp.dot(x, m2, preferred_element_type=jnp.float32)


def _dense(features, C, W, V):
    return pl.pallas_call(
        _dense_body,
        grid=(N // _ROWS,),
        in_specs=[
            pl.BlockSpec((_ROWS, D), lambda i: (i, 0)),
            pl.BlockSpec((D, H), lambda i: (0, 0)),
            pl.BlockSpec((2 * H, H), lambda i: (0, 0)),
            pl.BlockSpec((H, K), lambda i: (0, 0)),
        ],
        out_specs=[
            pl.BlockSpec((_ROWS, K), lambda i: (i, 0)),
            pl.BlockSpec((_ROWS, K), lambda i: (i, 0)),
        ],
        out_shape=[
            jax.ShapeDtypeStruct((N, K), jnp.float32),
            jax.ShapeDtypeStruct((N, K), jnp.float32),
        ],
    )(features, C, W, V)


def _sc_body(a1_hbm, a2_hbm, n1_hbm, n2_hbm, out_hbm,
             idx1, idx2, buf1, buf2, sem1, sem2):
    wid = lax.axis_index("s") * _NC + lax.axis_index("c")
    base = wid * _RPW
    pltpu.sync_copy(n1_hbm.at[pl.ds(base, _RPW)], idx1)
    pltpu.sync_copy(n2_hbm.at[pl.ds(base, _RPW)], idx2)
    cp1 = pltpu.async_copy(a1_hbm.at[idx1], buf1, sem1)
    cp2 = pltpu.async_copy(a2_hbm.at[idx2], buf2, sem2)
    cp1.wait()
    cp2.wait()

    def body(i, carry):
        buf1[i, :] = buf1[i, :] + buf2[i, :]
        return carry

    lax.fori_loop(0, _RPW, body, 0)
    pltpu.sync_copy(buf1, out_hbm.at[pl.ds(base, _RPW)])


def _sc_gather(a1, a2, n1, n2):
    mesh = plsc.VectorSubcoreMesh(core_axis_name="c", subcore_axis_name="s")
    f = pl.kernel(
        _sc_body,
        out_type=jax.ShapeDtypeStruct((_NPAD, K), jnp.float32),
        mesh=mesh,
        scratch_types=[
            pltpu.VMEM((_RPW,), jnp.int32),
            pltpu.VMEM((_RPW,), jnp.int32),
            pltpu.VMEM((_RPW, K), jnp.float32),
            pltpu.VMEM((_RPW, K), jnp.float32),
            pltpu.SemaphoreType.DMA,
            pltpu.SemaphoreType.DMA,
        ],
    )
    return f(a1, a2, n1, n2)


def _smax_body(t_ref, o_ref):
    t = t_ref[...]
    t = jnp.maximum(t, ALPHA * t)
    m = jnp.max(t, axis=1, keepdims=True)
    e = jnp.exp(t - m)
    s = jnp.sum(e, axis=1, keepdims=True)
    o_ref[...] = t - m - jnp.log(s)


def _smax(t):
    return pl.pallas_call(
        _smax_body,
        grid=(N // _ROWS,),
        in_specs=[pl.BlockSpec((_ROWS, K), lambda i: (i, 0))],
        out_specs=pl.BlockSpec((_ROWS, K), lambda i: (i, 0)),
        out_shape=jax.ShapeDtypeStruct((N, K), jnp.float32),
    )(t)


def kernel(features, C, W, V, nbr):
    a1, a2 = _dense(features, C, W, V)
    nbr_p = jnp.concatenate(
        [nbr, jnp.zeros((_NPAD - N, 2), jnp.int32)], axis=0)
    t = _sc_gather(a1, a2, nbr_p[:, 0], nbr_p[:, 1])
    return _smax(t)


# speedup vs baseline: 2.0987x; 2.0987x over previous
import jax
import jax.numpy as jnp
from jax import lax
from jax.experimental import pallas as pl
from jax.experimental.pallas import tpu as pltpu
from jax.experimental.pallas import tpu_sc as plsc

N = 100000
D = 128
H = 128
K = 16
ALPHA = 0.2

_NC, _NS = 2, 16
_NW = _NC * _NS
_RPW = 3128
_NPAD = _NW * _RPW

_ROWS = 5000


def _dense_body(f_ref, c_ref, w_ref, v_ref, a1_ref, a2_ref):
    wv = jnp.dot(w_ref[...], v_ref[...], preferred_element_type=jnp.float32)
    m1 = jnp.dot(c_ref[...], wv[:H], preferred_element_type=jnp.float32)
    m2 = jnp.dot(c_ref[...], wv[H:], preferred_element_type=jnp.float32)
    x = f_ref[...]
    a1_ref[...] = jnp.dot(x, m1, preferred_element_type=jnp.float32)
    a2_ref[...] = jnp.dot(x, m2, preferred_element_type=jnp.float32)


def _dense(features, C, W, V):
    return pl.pallas_call(
        _dense_body,
        grid=(N // _ROWS,),
        in_specs=[
            pl.BlockSpec((_ROWS, D), lambda i: (i, 0)),
            pl.BlockSpec((D, H), lambda i: (0, 0)),
            pl.BlockSpec((2 * H, H), lambda i: (0, 0)),
            pl.BlockSpec((H, K), lambda i: (0, 0)),
        ],
        out_specs=[
            pl.BlockSpec((_ROWS, K), lambda i: (i, 0)),
            pl.BlockSpec((_ROWS, K), lambda i: (i, 0)),
        ],
        out_shape=[
            jax.ShapeDtypeStruct((N, K), jnp.float32),
            jax.ShapeDtypeStruct((N, K), jnp.float32),
        ],
    )(features, C, W, V)


def _sc_body(a1_hbm, a2_hbm, n1_hbm, n2_hbm, out_hbm,
             idx1, idx2, buf1, buf2, sem1, sem2):
    wid = lax.axis_index("s") * _NC + lax.axis_index("c")
    base = wid * _RPW
    pltpu.sync_copy(n1_hbm.at[pl.ds(base, _RPW)], idx1)
    pltpu.sync_copy(n2_hbm.at[pl.ds(base, _RPW)], idx2)
    cp1 = pltpu.async_copy(a1_hbm.at[idx1], buf1, sem1)
    cp2 = pltpu.async_copy(a2_hbm.at[idx2], buf2, sem2)
    cp1.wait()
    cp2.wait()

    def body(i, carry):
        buf1[i, :] = buf1[i, :] + buf2[i, :]
        return carry

    lax.fori_loop(0, _RPW, body, 0)
    pltpu.sync_copy(buf1, out_hbm.at[pl.ds(base, _RPW)])


def _sc_gather(a1, a2, n1, n2):
    mesh = plsc.VectorSubcoreMesh(core_axis_name="c", subcore_axis_name="s")
    f = pl.kernel(
        _sc_body,
        out_type=jax.ShapeDtypeStruct((_NPAD, K), jnp.float32),
        mesh=mesh,
        compiler_params=pltpu.CompilerParams(use_tc_tiling_on_sc=False),
        scratch_types=[
            pltpu.VMEM((_RPW,), jnp.int32),
            pltpu.VMEM((_RPW,), jnp.int32),
            pltpu.VMEM((_RPW, K), jnp.float32),
            pltpu.VMEM((_RPW, K), jnp.float32),
            pltpu.SemaphoreType.DMA,
            pltpu.SemaphoreType.DMA,
        ],
    )
    return f(a1, a2, n1, n2)


def _smax_body(t_ref, o_ref):
    t = t_ref[...]
    t = jnp.maximum(t, ALPHA * t)
    m = jnp.max(t, axis=1, keepdims=True)
    e = jnp.exp(t - m)
    s = jnp.sum(e, axis=1, keepdims=True)
    o_ref[...] = t - m - jnp.log(s)


def _smax(t):
    return pl.pallas_call(
        _smax_body,
        grid=(N // _ROWS,),
        in_specs=[pl.BlockSpec((_ROWS, K), lambda i: (i, 0))],
        out_specs=pl.BlockSpec((_ROWS, K), lambda i: (i, 0)),
        out_shape=jax.ShapeDtypeStruct((N, K), jnp.float32),
    )(t)


def kernel(features, C, W, V, nbr):
    a1, a2 = _dense(features, C, W, V)
    nbr_p = jnp.concatenate(
        [nbr, jnp.zeros((_NPAD - N, 2), jnp.int32)], axis=0)
    t = _sc_gather(a1, a2, nbr_p[:, 0], nbr_p[:, 1])
    return _smax(t)


# speedup vs baseline: 2.3912x; 1.1394x over previous
import jax
import jax.numpy as jnp
from jax import lax
from jax.experimental import pallas as pl
from jax.experimental.pallas import tpu as pltpu
from jax.experimental.pallas import tpu_sc as plsc

N = 100000
D = 128
H = 128
K = 16
ALPHA = 0.2

_NC, _NS = 2, 16
_NW = _NC * _NS
_RPW = 3128
_NPAD = _NW * _RPW
_GPAD = _NPAD // 8

_ROWS = 6400
_GR = _ROWS // 8

_SROWS = 6256
_SN = _SROWS * 8


def _dense_body(f_ref, c_ref, w_ref, v_ref, a1_ref, a2_ref):
    wv = jnp.dot(w_ref[...], v_ref[...], preferred_element_type=jnp.float32)
    m1 = jnp.dot(c_ref[...], wv[:H], preferred_element_type=jnp.float32)
    m2 = jnp.dot(c_ref[...], wv[H:], preferred_element_type=jnp.float32)
    x = f_ref[...]
    a1_ref[...] = jnp.dot(x, m1, preferred_element_type=jnp.float32)
    a2_ref[...] = jnp.dot(x, m2, preferred_element_type=jnp.float32)


def _dense(features, C, W, V):
    return pl.pallas_call(
        _dense_body,
        grid=(pl.cdiv(N, _ROWS),),
        in_specs=[
            pl.BlockSpec((_ROWS, D), lambda i: (i, 0)),
            pl.BlockSpec((D, H), lambda i: (0, 0)),
            pl.BlockSpec((2 * H, H), lambda i: (0, 0)),
            pl.BlockSpec((H, K), lambda i: (0, 0)),
        ],
        out_specs=[
            pl.BlockSpec((_ROWS, K), lambda i: (i, 0)),
            pl.BlockSpec((_ROWS, K), lambda i: (i, 0)),
        ],
        out_shape=[
            jax.ShapeDtypeStruct((_NPAD, K), jnp.float32),
            jax.ShapeDtypeStruct((_NPAD, K), jnp.float32),
        ],
    )(features, C, W, V)


def _sc_body(a1_hbm, a2_hbm, n1_hbm, n2_hbm, out_hbm,
             idx1, idx2, buf1, buf2, sem1, sem2):
    wid = lax.axis_index("s") * _NC + lax.axis_index("c")
    base = wid * _RPW
    pltpu.sync_copy(n1_hbm.at[pl.ds(base, _RPW)], idx1)
    pltpu.sync_copy(n2_hbm.at[pl.ds(base, _RPW)], idx2)
    cp1 = pltpu.async_copy(a1_hbm.at[idx1], buf1, sem1)
    cp2 = pltpu.async_copy(a2_hbm.at[idx2], buf2, sem2)
    cp1.wait()
    cp2.wait()

    def body(i, carry):
        buf1[i, :] = buf1[i, :] + buf2[i, :]
        return carry

    lax.fori_loop(0, _RPW, body, 0)
    pltpu.sync_copy(buf1, out_hbm.at[pl.ds(base, _RPW)])


def _sc_gather(a1, a2, n1, n2):
    mesh = plsc.VectorSubcoreMesh(core_axis_name="c", subcore_axis_name="s")
    f = pl.kernel(
        _sc_body,
        out_type=jax.ShapeDtypeStruct((_NPAD, K), jnp.float32),
        mesh=mesh,
        compiler_params=pltpu.CompilerParams(use_tc_tiling_on_sc=False),
        scratch_types=[
            pltpu.VMEM((_RPW,), jnp.int32),
            pltpu.VMEM((_RPW,), jnp.int32),
            pltpu.VMEM((_RPW, K), jnp.float32),
            pltpu.VMEM((_RPW, K), jnp.float32),
            pltpu.SemaphoreType.DMA,
            pltpu.SemaphoreType.DMA,
        ],
    )
    return f(a1, a2, n1, n2)


def _smax_body(t_ref, o_ref):
    t = t_ref[...]
    t = jnp.maximum(t, ALPHA * t)
    m = jnp.max(t, axis=1, keepdims=True)
    e = jnp.exp(t - m)
    s = jnp.sum(e, axis=1, keepdims=True)
    r = t - m - jnp.log(s)
    o_ref[...] = r.T


_SR = 12800


def _smax(t):
    return pl.pallas_call(
        _smax_body,
        grid=(pl.cdiv(N, _SR),),
        in_specs=[pl.BlockSpec((_SR, K), lambda i: (i, 0))],
        out_specs=pl.BlockSpec((K, _SR), lambda i: (0, i)),
        out_shape=jax.ShapeDtypeStruct((K, N), jnp.float32),
    )(t)


def kernel(features, C, W, V, nbr):
    a1, a2 = _dense(features, C, W, V)
    nbr_p = jnp.concatenate(
        [nbr, jnp.zeros((_NPAD - N, 2), jnp.int32)], axis=0)
    t = _sc_gather(a1, a2, nbr_p[:, 0], nbr_p[:, 1])
    ot = _smax(t)
    return jnp.transpose(ot)


# speedup vs baseline: 2.4029x; 1.0049x over previous
import jax
import jax.numpy as jnp
from jax import lax
from jax.experimental import pallas as pl
from jax.experimental.pallas import tpu as pltpu
from jax.experimental.pallas import tpu_sc as plsc

N = 100000
D = 128
H = 128
K = 16
ALPHA = 0.2

_NC, _NS = 2, 16
_NW = _NC * _NS
_RPW = 3128
_NPAD = _NW * _RPW

_CH = 184
_NCH = _RPW // _CH

_ROWS = 6400


def _dense_body(f_ref, c_ref, w_ref, v_ref, a1_ref, a2_ref):
    wv = jnp.dot(w_ref[...], v_ref[...], preferred_element_type=jnp.float32)
    m1 = jnp.dot(c_ref[...], wv[:H], preferred_element_type=jnp.float32)
    m2 = jnp.dot(c_ref[...], wv[H:], preferred_element_type=jnp.float32)
    x = f_ref[...]
    a1_ref[...] = jnp.dot(x, m1, preferred_element_type=jnp.float32)
    a2_ref[...] = jnp.dot(x, m2, preferred_element_type=jnp.float32)


def _dense(features, C, W, V):
    return pl.pallas_call(
        _dense_body,
        grid=(pl.cdiv(N, _ROWS),),
        in_specs=[
            pl.BlockSpec((_ROWS, D), lambda i: (i, 0)),
            pl.BlockSpec((D, H), lambda i: (0, 0)),
            pl.BlockSpec((2 * H, H), lambda i: (0, 0)),
            pl.BlockSpec((H, K), lambda i: (0, 0)),
        ],
        out_specs=[
            pl.BlockSpec((_ROWS, K), lambda i: (i, 0)),
            pl.BlockSpec((_ROWS, K), lambda i: (i, 0)),
        ],
        out_shape=[
            jax.ShapeDtypeStruct((_NPAD, K), jnp.float32),
            jax.ShapeDtypeStruct((_NPAD, K), jnp.float32),
        ],
    )(features, C, W, V)


def _sc_body(a1_hbm, a2_hbm, n1_hbm, n2_hbm, out_hbm,
             idx1, idx2, b1a, b1b, b2a, b2b, semi, sem1, sem2, semo):
    wid = lax.axis_index("s") * _NC + lax.axis_index("c")
    base = wid * _RPW
    ci1 = pltpu.async_copy(n1_hbm.at[pl.ds(base, _RPW)], idx1, semi)
    ci2 = pltpu.async_copy(n2_hbm.at[pl.ds(base, _RPW)], idx2, semi)
    ci1.wait()
    ci2.wait()

    def start(k, slot):
        sl = pl.ds(k * _CH, _CH)
        c1 = pltpu.async_copy(a1_hbm.at[idx1.at[sl]], [b1a, b1b][slot], sem1)
        c2 = pltpu.async_copy(a2_hbm.at[idx2.at[sl]], [b2a, b2b][slot], sem2)
        return c1, c2

    outs = []
    pend = start(0, 0)
    for k in range(_NCH):
        slot = k % 2
        nxt = start(k + 1, (k + 1) % 2) if k + 1 < _NCH else None
        pend[0].wait()
        pend[1].wait()
        b1s = [b1a, b1b][slot]
        b2s = [b2a, b2b][slot]

        def body(i, carry):
            b1s[i, :] = b1s[i, :] + b2s[i, :]
            return carry

        lax.fori_loop(0, _CH, body, 0)
        outs.append(pltpu.async_copy(
            b1s, out_hbm.at[pl.ds(base + k * _CH, _CH)], semo))
        if len(outs) >= 2:
            outs.pop(0).wait()
        pend = nxt
    for cp in outs:
        cp.wait()


def _sc_gather(a1, a2, n1, n2):
    mesh = plsc.VectorSubcoreMesh(core_axis_name="c", subcore_axis_name="s")
    f = pl.kernel(
        _sc_body,
        out_type=jax.ShapeDtypeStruct((_NPAD, K), jnp.float32),
        mesh=mesh,
        compiler_params=pltpu.CompilerParams(use_tc_tiling_on_sc=False),
        scratch_types=[
            pltpu.VMEM((_RPW,), jnp.int32),
            pltpu.VMEM((_RPW,), jnp.int32),
            pltpu.VMEM((_CH, K), jnp.float32),
            pltpu.VMEM((_CH, K), jnp.float32),
            pltpu.VMEM((_CH, K), jnp.float32),
            pltpu.VMEM((_CH, K), jnp.float32),
            pltpu.SemaphoreType.DMA,
            pltpu.SemaphoreType.DMA,
            pltpu.SemaphoreType.DMA,
            pltpu.SemaphoreType.DMA,
        ],
    )
    return f(a1, a2, n1, n2)


_SR = 12800


def _smax_body(t_ref, o_ref):
    t = t_ref[...]
    t = jnp.maximum(t, ALPHA * t)
    m = jnp.max(t, axis=1, keepdims=True)
    e = jnp.exp(t - m)
    s = jnp.sum(e, axis=1, keepdims=True)
    r = t - m - jnp.log(s)
    o_ref[...] = r.T


def _smax(t):
    return pl.pallas_call(
        _smax_body,
        grid=(pl.cdiv(N, _SR),),
        in_specs=[pl.BlockSpec((_SR, K), lambda i: (i, 0))],
        out_specs=pl.BlockSpec((K, _SR), lambda i: (0, i)),
        out_shape=jax.ShapeDtypeStruct((K, N), jnp.float32),
    )(t)


def kernel(features, C, W, V, nbr):
    a1, a2 = _dense(features, C, W, V)
    nbr_p = jnp.concatenate(
        [nbr, jnp.zeros((_NPAD - N, 2), jnp.int32)], axis=0)
    t = _sc_gather(a1, a2, nbr_p[:, 0], nbr_p[:, 1])
    ot = _smax(t)
    return jnp.transpose(ot)


# speedup vs baseline: 2.9445x; 1.2254x over previous
import jax
import jax.numpy as jnp
from jax import lax
from jax.experimental import pallas as pl
from jax.experimental.pallas import tpu as pltpu
from jax.experimental.pallas import tpu_sc as plsc

N = 100000
D = 128
H = 128
K = 16
ALPHA = 0.2

_NC, _NS = 2, 16
_NW = _NC * _NS
_RPW = 3128
_NPAD = _NW * _RPW

_CH = 184
_NCH = _RPW // _CH

_ROWS = 6400


def _dense_body(f_ref, c_ref, w_ref, v_ref, a_ref):
    wv = jnp.dot(w_ref[...], v_ref[...], preferred_element_type=jnp.float32)
    m1 = jnp.dot(c_ref[...], wv[:H], preferred_element_type=jnp.float32)
    m2 = jnp.dot(c_ref[...], wv[H:], preferred_element_type=jnp.float32)
    m = jnp.concatenate([m1, m2], axis=1)
    x = f_ref[...]
    a_ref[...] = jnp.dot(x, m, preferred_element_type=jnp.float32)


def _dense(features, C, W, V):
    return pl.pallas_call(
        _dense_body,
        grid=(pl.cdiv(N, _ROWS),),
        in_specs=[
            pl.BlockSpec((_ROWS, D), lambda i: (i, 0)),
            pl.BlockSpec((D, H), lambda i: (0, 0)),
            pl.BlockSpec((2 * H, H), lambda i: (0, 0)),
            pl.BlockSpec((H, K), lambda i: (0, 0)),
        ],
        out_specs=pl.BlockSpec((_ROWS, 2 * K), lambda i: (i, 0)),
        out_shape=jax.ShapeDtypeStruct((_NPAD, 2 * K), jnp.float32),
    )(features, C, W, V)


def _sc_body(ac_hbm, n1_hbm, n2_hbm, out_hbm,
             idx1, idx2, b1a, b1b, b2a, b2b, semi, sem1, sem2, semo):
    wid = lax.axis_index("s") * _NC + lax.axis_index("c")
    base = wid * _RPW
    ci1 = pltpu.async_copy(n1_hbm.at[pl.ds(base, _RPW)], idx1, semi)
    ci2 = pltpu.async_copy(n2_hbm.at[pl.ds(base, _RPW)], idx2, semi)
    ci1.wait()
    ci2.wait()

    def start(k, slot):
        sl = pl.ds(k * _CH, _CH)
        c1 = pltpu.async_copy(ac_hbm.at[idx1.at[sl]], [b1a, b1b][slot], sem1)
        c2 = pltpu.async_copy(ac_hbm.at[idx2.at[sl]], [b2a, b2b][slot], sem2)
        return c1, c2

    outs = []
    pend = start(0, 0)
    for k in range(_NCH):
        slot = k % 2
        nxt = start(k + 1, (k + 1) % 2) if k + 1 < _NCH else None
        pend[0].wait()
        pend[1].wait()
        b1s = [b1a, b1b][slot]
        b2s = [b2a, b2b][slot]

        def body(i, carry):
            b1s[i, :] = b1s[i, :] + b2s[i, :]
            return carry

        lax.fori_loop(0, _CH, body, 0)
        outs.append(pltpu.async_copy(
            b1s, out_hbm.at[pl.ds(base + k * _CH, _CH)], semo))
        if len(outs) >= 2:
            outs.pop(0).wait()
        pend = nxt
    for cp in outs:
        cp.wait()


def _sc_gather(acv, n1, n2):
    mesh = plsc.VectorSubcoreMesh(core_axis_name="c", subcore_axis_name="s")
    f = pl.kernel(
        _sc_body,
        out_type=jax.ShapeDtypeStruct((_NPAD, K), jnp.float32),
        mesh=mesh,
        compiler_params=pltpu.CompilerParams(use_tc_tiling_on_sc=False),
        scratch_types=[
            pltpu.VMEM((_RPW,), jnp.int32),
            pltpu.VMEM((_RPW,), jnp.int32),
            pltpu.VMEM((_CH, K), jnp.float32),
            pltpu.VMEM((_CH, K), jnp.float32),
            pltpu.VMEM((_CH, K), jnp.float32),
            pltpu.VMEM((_CH, K), jnp.float32),
            pltpu.SemaphoreType.DMA,
            pltpu.SemaphoreType.DMA,
            pltpu.SemaphoreType.DMA,
            pltpu.SemaphoreType.DMA,
        ],
    )
    return f(acv, n1, n2)


_SR = 12800


def _smax_body(t_ref, o_ref):
    t = t_ref[...]
    t = jnp.maximum(t, ALPHA * t)
    m = jnp.max(t, axis=1, keepdims=True)
    e = jnp.exp(t - m)
    s = jnp.sum(e, axis=1, keepdims=True)
    r = t - m - jnp.log(s)
    o_ref[...] = r.T


def _smax(t):
    return pl.pallas_call(
        _smax_body,
        grid=(pl.cdiv(N, _SR),),
        in_specs=[pl.BlockSpec((_SR, K), lambda i: (i, 0))],
        out_specs=pl.BlockSpec((K, _SR), lambda i: (0, i)),
        out_shape=jax.ShapeDtypeStruct((K, N), jnp.float32),
    )(t)


def kernel(features, C, W, V, nbr):
    ac = _dense(features, C, W, V)
    acv = jnp.reshape(ac, (2 * _NPAD, K))
    nbr_p = jnp.concatenate(
        [nbr, jnp.zeros((_NPAD - N, 2), jnp.int32)], axis=0)
    t = _sc_gather(acv, nbr_p[:, 0] * 2, nbr_p[:, 1] * 2 + 1)
    ot = _smax(t)
    return jnp.transpose(ot)


# speedup vs baseline: 3.8612x; 1.3113x over previous
import jax
import jax.numpy as jnp
from jax import lax
from jax.experimental import pallas as pl
from jax.experimental.pallas import tpu as pltpu
from jax.experimental.pallas import tpu_sc as plsc

N = 100000
D = 128
H = 128
K = 16
ALPHA = 0.2

_NC, _NS = 2, 16
_NW = _NC * _NS
_RPW = 3128
_NPAD = _NW * _RPW

_CH = 184
_NCH = _RPW // _CH

_ROWS = 6400


def _dense_body(f_ref, c_ref, w_ref, v_ref, a_ref):
    wv = jnp.dot(w_ref[...], v_ref[...], preferred_element_type=jnp.float32)
    m1 = jnp.dot(c_ref[...], wv[:H], preferred_element_type=jnp.float32)
    m2 = jnp.dot(c_ref[...], wv[H:], preferred_element_type=jnp.float32)
    m = jnp.concatenate([m1, m2], axis=1)
    x = f_ref[...]
    a_ref[...] = jnp.dot(x, m, preferred_element_type=jnp.float32)


def _dense(features, C, W, V):
    return pl.pallas_call(
        _dense_body,
        grid=(pl.cdiv(N, _ROWS),),
        in_specs=[
            pl.BlockSpec((_ROWS, D), lambda i: (i, 0)),
            pl.BlockSpec((D, H), lambda i: (0, 0)),
            pl.BlockSpec((2 * H, H), lambda i: (0, 0)),
            pl.BlockSpec((H, K), lambda i: (0, 0)),
        ],
        out_specs=pl.BlockSpec((_ROWS, 2 * K), lambda i: (i, 0)),
        out_shape=jax.ShapeDtypeStruct((_NPAD, 2 * K), jnp.float32),
    )(features, C, W, V)


def _sc_body(ac_hbm, n1_hbm, n2_hbm, out_hbm,
             idx1, idx2, b1a, b1b, b2a, b2b, semi, sem1, sem2, semo):
    wid = lax.axis_index("s") * _NC + lax.axis_index("c")
    base = wid * _RPW
    j0 = wid // 4
    g0 = (wid % 4) * _RPW
    ci1 = pltpu.async_copy(n1_hbm.at[pl.ds(base, _RPW)], idx1, semi)
    ci2 = pltpu.async_copy(n2_hbm.at[pl.ds(base, _RPW)], idx2, semi)
    ci1.wait()
    ci2.wait()

    def start(k, slot):
        sl = pl.ds(k * _CH, _CH)
        c1 = pltpu.async_copy(ac_hbm.at[idx1.at[sl]], [b1a, b1b][slot], sem1)
        c2 = pltpu.async_copy(ac_hbm.at[idx2.at[sl]], [b2a, b2b][slot], sem2)
        return c1, c2

    outs = []
    pend = start(0, 0)
    for k in range(_NCH):
        slot = k % 2
        nxt = start(k + 1, (k + 1) % 2) if k + 1 < _NCH else None
        pend[0].wait()
        pend[1].wait()
        b1s = [b1a, b1b][slot]
        b2s = [b2a, b2b][slot]

        def body(i, carry):
            b1s[i, :] = b1s[i, :] + b2s[i, :]
            return carry

        lax.fori_loop(0, _CH, body, 0)
        outs.append(pltpu.async_copy(
            b1s, out_hbm.at[pl.ds(g0 + k * _CH, _CH), pl.ds(K * j0, K)], semo))
        if len(outs) >= 2:
            outs.pop(0).wait()
        pend = nxt
    for cp in outs:
        cp.wait()


def _sc_gather(acv, n1, n2):
    mesh = plsc.VectorSubcoreMesh(core_axis_name="c", subcore_axis_name="s")
    f = pl.kernel(
        _sc_body,
        out_type=jax.ShapeDtypeStruct((_NPAD // 8, 8 * K), jnp.float32),
        mesh=mesh,
        compiler_params=pltpu.CompilerParams(use_tc_tiling_on_sc=False),
        scratch_types=[
            pltpu.VMEM((_RPW,), jnp.int32),
            pltpu.VMEM((_RPW,), jnp.int32),
            pltpu.VMEM((_CH, K), jnp.float32),
            pltpu.VMEM((_CH, K), jnp.float32),
            pltpu.VMEM((_CH, K), jnp.float32),
            pltpu.VMEM((_CH, K), jnp.float32),
            pltpu.SemaphoreType.DMA,
            pltpu.SemaphoreType.DMA,
            pltpu.SemaphoreType.DMA,
            pltpu.SemaphoreType.DMA,
        ],
    )
    return f(acv, n1, n2)


_GPAD = _NPAD // 8
_SHIFT = 20.0


def _smax_body(tp_ref, o_ref):
    p = tp_ref[...]
    p = jnp.maximum(p, ALPHA * p)
    li = lax.broadcasted_iota(jnp.int32, (128, 128), 0) // K
    lj = lax.broadcasted_iota(jnp.int32, (128, 128), 1) // K
    g = (li == lj).astype(jnp.float32)
    e = jnp.exp(p - _SHIFT)
    s = jnp.dot(e, g, preferred_element_type=jnp.float32)
    r = p - _SHIFT - jnp.log(s)
    o_ref[...] = r.T


def _smax(tp):
    return pl.pallas_call(
        _smax_body,
        grid=(1,),
        in_specs=[pl.BlockSpec((_GPAD, 128), lambda i: (0, 0))],
        out_specs=pl.BlockSpec((128, _GPAD), lambda i: (0, 0)),
        out_shape=jax.ShapeDtypeStruct((128, _GPAD), jnp.float32),
    )(tp)


def kernel(features, C, W, V, nbr):
    ac = _dense(features, C, W, V)
    acv = jnp.reshape(ac, (2 * _NPAD, K))
    nbr_p = jnp.concatenate(
        [nbr, jnp.zeros((_NPAD - N, 2), jnp.int32)], axis=0)
    tp = _sc_gather(acv, nbr_p[:, 0] * 2, nbr_p[:, 1] * 2 + 1)
    rt = _smax(tp)
    ot = jnp.swapaxes(rt.reshape(8, K, _GPAD), 0, 1).reshape(K, _NPAD)[:, :N]
    return jnp.transpose(ot)


# speedup vs baseline: 5.0014x; 1.2953x over previous
import jax
import jax.numpy as jnp
from jax import lax
from jax.experimental import pallas as pl
from jax.experimental.pallas import tpu as pltpu
from jax.experimental.pallas import tpu_sc as plsc

N = 100000
D = 128
H = 128
K = 16
ALPHA = 0.2

_NC, _NS = 2, 16
_NW = _NC * _NS
_RPW = 3128
_NPAD = _NW * _RPW

_CH = 184
_NCH = _RPW // _CH

_ROWS = 6400


def _dense_body(f_ref, c_ref, w_ref, v_ref, a_ref):
    wv = jnp.dot(w_ref[...], v_ref[...], preferred_element_type=jnp.float32)
    m1 = jnp.dot(c_ref[...], wv[:H], preferred_element_type=jnp.float32)
    m2 = jnp.dot(c_ref[...], wv[H:], preferred_element_type=jnp.float32)
    m = jnp.concatenate(
        [m1, m2, jnp.zeros((H, 128 - 2 * K), jnp.float32)], axis=1)
    x = f_ref[...]
    a_ref[...] = jnp.dot(x, m, preferred_element_type=jnp.float32)


def _dense(features, C, W, V):
    return pl.pallas_call(
        _dense_body,
        grid=(pl.cdiv(N, _ROWS),),
        in_specs=[
            pl.BlockSpec((_ROWS, D), lambda i: (i, 0)),
            pl.BlockSpec((D, H), lambda i: (0, 0)),
            pl.BlockSpec((2 * H, H), lambda i: (0, 0)),
            pl.BlockSpec((H, K), lambda i: (0, 0)),
        ],
        out_specs=pl.BlockSpec((_ROWS, 128), lambda i: (i, 0)),
        out_shape=jax.ShapeDtypeStruct((_NPAD, 128), jnp.float32),
    )(features, C, W, V)


def _sc_body(ac_hbm, n1_hbm, n2_hbm, out_hbm,
             idx1, idx2, b1a, b1b, b2a, b2b, semi, sem1, sem2, semo):
    wid = lax.axis_index("s") * _NC + lax.axis_index("c")
    base = wid * _RPW
    j0 = wid // 4
    g0 = (wid % 4) * _RPW
    ci1 = pltpu.async_copy(n1_hbm.at[pl.ds(base, _RPW)], idx1, semi)
    ci2 = pltpu.async_copy(n2_hbm.at[pl.ds(base, _RPW)], idx2, semi)
    ci1.wait()
    ci2.wait()

    def start(k, slot):
        sl = pl.ds(k * _CH, _CH)
        c1 = pltpu.async_copy(ac_hbm.at[idx1.at[sl]], [b1a, b1b][slot], sem1)
        c2 = pltpu.async_copy(ac_hbm.at[idx2.at[sl]], [b2a, b2b][slot], sem2)
        return c1, c2

    outs = []
    pend = start(0, 0)
    for k in range(_NCH):
        slot = k % 2
        nxt = start(k + 1, (k + 1) % 2) if k + 1 < _NCH else None
        pend[0].wait()
        pend[1].wait()
        b1s = [b1a, b1b][slot]
        b2s = [b2a, b2b][slot]

        def body(i, carry):
            b1s[i, :] = b1s[i, :] + b2s[i, :]
            return carry

        lax.fori_loop(0, _CH, body, 0)
        outs.append(pltpu.async_copy(
            b1s, out_hbm.at[pl.ds(g0 + k * _CH, _CH), pl.ds(K * j0, K)], semo))
        if len(outs) >= 2:
            outs.pop(0).wait()
        pend = nxt
    for cp in outs:
        cp.wait()


def _sc_gather(acv, n1, n2):
    mesh = plsc.VectorSubcoreMesh(core_axis_name="c", subcore_axis_name="s")
    f = pl.kernel(
        _sc_body,
        out_type=jax.ShapeDtypeStruct((_NPAD // 8, 8 * K), jnp.float32),
        mesh=mesh,
        compiler_params=pltpu.CompilerParams(use_tc_tiling_on_sc=False),
        scratch_types=[
            pltpu.VMEM((_RPW,), jnp.int32),
            pltpu.VMEM((_RPW,), jnp.int32),
            pltpu.VMEM((_CH, K), jnp.float32),
            pltpu.VMEM((_CH, K), jnp.float32),
            pltpu.VMEM((_CH, K), jnp.float32),
            pltpu.VMEM((_CH, K), jnp.float32),
            pltpu.SemaphoreType.DMA,
            pltpu.SemaphoreType.DMA,
            pltpu.SemaphoreType.DMA,
            pltpu.SemaphoreType.DMA,
        ],
    )
    return f(acv, n1, n2)


_GPAD = _NPAD // 8
_SHIFT = 20.0


def _smax_body(tp_ref, o_ref):
    p = tp_ref[...]
    p = jnp.maximum(p, ALPHA * p)
    li = lax.broadcasted_iota(jnp.int32, (128, 128), 0) // K
    lj = lax.broadcasted_iota(jnp.int32, (128, 128), 1) // K
    g = (li == lj).astype(jnp.float32)
    e = jnp.exp(p - _SHIFT)
    s = jnp.dot(e, g, preferred_element_type=jnp.float32)
    r = p - _SHIFT - jnp.log(s)
    o_ref[...] = r.T


def _smax(tp):
    return pl.pallas_call(
        _smax_body,
        grid=(1,),
        in_specs=[pl.BlockSpec((_GPAD, 128), lambda i: (0, 0))],
        out_specs=pl.BlockSpec((128, _GPAD), lambda i: (0, 0)),
        out_shape=jax.ShapeDtypeStruct((128, _GPAD), jnp.float32),
    )(tp)


def kernel(features, C, W, V, nbr):
    ac = _dense(features, C, W, V)
    acv = jnp.reshape(ac, (8 * _NPAD, K))
    nbr_p = jnp.concatenate(
        [nbr, jnp.zeros((_NPAD - N, 2), jnp.int32)], axis=0)
    tp = _sc_gather(acv, nbr_p[:, 0] * 8, nbr_p[:, 1] * 8 + 1)
    rt = _smax(tp)
    ot = jnp.swapaxes(rt.reshape(8, K, _GPAD), 0, 1).reshape(K, _NPAD)[:, :N]
    return jnp.transpose(ot)
